# Initial kernel scaffold; baseline (speedup 1.0000x reference)
import jax
import jax.numpy as jnp
from jax.experimental import pallas as pl


def kernel(input_ids, token_type_ids, position_ids, zixing_ids, W_word, W_pos, W_tok, W_glyph, W_gfc, b_gfc, W_map, b_map, ln_g, ln_b):
    raise NotImplementedError("write your pallas kernel here")


# speedup vs baseline: 5.6252x; 5.6252x over previous
import functools

import jax
import jax.numpy as jnp
from jax import lax
from jax.experimental import pallas as pl
from jax.experimental.pallas import tpu as pltpu
from jax.experimental.pallas import tpu_sc as plsc

LN_EPS = 1e-12


def _word_transform_body(w_ref, a_ref, o_ref):
    o_ref[...] = jnp.dot(w_ref[...], a_ref[...],
                         preferred_element_type=jnp.float32)


def _glyph_transform_body(wg_ref, wgfc_ref, g2_ref, bg_ref, bm_ref,
                          pos_ref, wpos_ref, og_ref, oposb_ref):
    c = jnp.dot(wgfc_ref[...], g2_ref[...], preferred_element_type=jnp.float32)
    og_ref[...] = jnp.dot(wg_ref[...], c, preferred_element_type=jnp.float32)
    bias = jnp.dot(bg_ref[...], g2_ref[...],
                   preferred_element_type=jnp.float32) + bm_ref[...]
    pos = pos_ref[0, 0, :]
    max_pos = wpos_ref.shape[0]
    oh = (pos[:, None] ==
          lax.broadcasted_iota(jnp.int32, (pos.shape[0], max_pos), 1)
          ).astype(jnp.float32)
    oposb_ref[...] = jnp.dot(oh, wpos_ref[...],
                             preferred_element_type=jnp.float32) + bias


def _post_ln_body(s_ref, t_ref, wtok_ref, posb_ref, g_ref, b_ref, o_ref):
    t = t_ref[:, 0, :].astype(jnp.float32)
    rb, s, h = s_ref.shape
    tf = lax.broadcast_in_dim(t, (rb, s, h), (0, 1))
    tok = wtok_ref[0, :] + tf * (wtok_ref[1, :] - wtok_ref[0, :])
    x = s_ref[...] + posb_ref[...][None] + tok
    mu = jnp.mean(x, axis=-1, keepdims=True)
    var = jnp.mean(jnp.square(x - mu), axis=-1, keepdims=True)
    y = (x - mu) * lax.rsqrt(var + LN_EPS)
    o_ref[...] = y * g_ref[0, :] + b_ref[0, :]


def kernel(input_ids, token_type_ids, position_ids, zixing_ids,
           W_word, W_pos, W_tok, W_glyph, W_gfc, b_gfc,
           W_map, b_map, ln_g, ln_b):
    B, S = input_ids.shape
    V, H = W_word.shape
    GV, _ = W_glyph.shape
    N = B * S
    A = W_map[:H]
    G2 = W_map[H:]

    WBLK = 2000
    W_word2 = pl.pallas_call(
        _word_transform_body,
        grid=(V // WBLK,),
        in_specs=[
            pl.BlockSpec((WBLK, H), lambda i: (i, 0)),
            pl.BlockSpec((H, H), lambda i: (0, 0)),
        ],
        out_specs=pl.BlockSpec((WBLK, H), lambda i: (i, 0)),
        out_shape=jax.ShapeDtypeStruct((V, H), jnp.float32),
    )(W_word, A)

    pos3 = position_ids.reshape(1, 1, S).astype(jnp.int32)
    W_glyph2, posb = pl.pallas_call(
        _glyph_transform_body,
        out_shape=(jax.ShapeDtypeStruct((GV, H), jnp.float32),
                   jax.ShapeDtypeStruct((S, H), jnp.float32)),
    )(W_glyph, W_gfc, G2, b_gfc.reshape(1, H), b_map.reshape(1, H),
      pos3, W_pos)

    info = plsc.get_sparse_core_info()
    NC, NS = info.num_cores, info.num_subcores
    NW = NC * NS
    per_w = N // NW
    K = 128
    n_chunks = per_w // K
    mesh = plsc.VectorSubcoreMesh(core_axis_name="c", subcore_axis_name="s")

    @functools.partial(
        pl.kernel,
        out_type=jax.ShapeDtypeStruct((N, H), jnp.float32),
        mesh=mesh,
        scratch_types=[
            pltpu.VMEM((K,), jnp.int32),
            pltpu.VMEM((K,), jnp.int32),
            pltpu.VMEM((K, H), jnp.float32),
            pltpu.SemaphoreType.DMA,
        ],
    )
    def _gather_sum(wt_hbm, gt_hbm, idw_hbm, idg_hbm, out_hbm,
                    idw_v, idg_v, rows_v, sem):
        wid = lax.axis_index("s") * NC + lax.axis_index("c")

        @pl.loop(0, n_chunks)
        def _chunk(i):
            base = (wid * n_chunks + i) * K
            pltpu.sync_copy(idw_hbm.at[pl.ds(base, K)], idw_v)
            pltpu.sync_copy(idg_hbm.at[pl.ds(base, K)], idg_v)
            pltpu.async_copy(wt_hbm.at[idw_v], rows_v, sem).wait()
            pltpu.async_copy(gt_hbm.at[idg_v], rows_v, sem, add=True).wait()
            pltpu.sync_copy(rows_v, out_hbm.at[pl.ds(base, K)])

    sums = _gather_sum(W_word2, W_glyph2,
                       input_ids.reshape(N), zixing_ids.reshape(N))

    RB = 8
    type3 = token_type_ids.reshape(B, 1, S).astype(jnp.int32)
    out = pl.pallas_call(
        _post_ln_body,
        grid=(B // RB,),
        in_specs=[
            pl.BlockSpec((RB, S, H), lambda i: (i, 0, 0)),
            pl.BlockSpec((RB, 1, S), lambda i: (i, 0, 0)),
            pl.BlockSpec((2, H), lambda i: (0, 0)),
            pl.BlockSpec((S, H), lambda i: (0, 0)),
            pl.BlockSpec((1, H), lambda i: (0, 0)),
            pl.BlockSpec((1, H), lambda i: (0, 0)),
        ],
        out_specs=pl.BlockSpec((RB, S, H), lambda i: (i, 0, 0)),
        out_shape=jax.ShapeDtypeStruct((B, S, H), jnp.float32),
    )(sums.reshape(B, S, H), type3, W_tok, posb,
      ln_g.reshape(1, H), ln_b.reshape(1, H))

    return out


# speedup vs baseline: 6.6477x; 1.1818x over previous
import functools

import jax
import jax.numpy as jnp
from jax import lax
from jax.experimental import pallas as pl
from jax.experimental.pallas import tpu as pltpu
from jax.experimental.pallas import tpu_sc as plsc

LN_EPS = 1e-12
NSPLIT = 2


def _word_transform_body(w_ref, a_ref, o_ref):
    o_ref[...] = jnp.dot(w_ref[...], a_ref[...],
                         preferred_element_type=jnp.float32)


def _glyph_transform_body(wg_ref, wgfc_ref, g2_ref, bg_ref, bm_ref,
                          pos_ref, wpos_ref, og_ref, oposb_ref):
    c = jnp.dot(wgfc_ref[...], g2_ref[...], preferred_element_type=jnp.float32)
    og_ref[...] = jnp.dot(wg_ref[...], c, preferred_element_type=jnp.float32)
    bias = jnp.dot(bg_ref[...], g2_ref[...],
                   preferred_element_type=jnp.float32) + bm_ref[...]
    pos = pos_ref[0, 0, :]
    max_pos = wpos_ref.shape[0]
    oh = (pos[:, None] ==
          lax.broadcasted_iota(jnp.int32, (pos.shape[0], max_pos), 1)
          ).astype(jnp.float32)
    oposb_ref[...] = jnp.dot(oh, wpos_ref[...],
                             preferred_element_type=jnp.float32) + bias


def _ln_math(s_raw, t_ref, wtok_ref, posb_ref, g_ref, b_ref):
    t = t_ref[:, 0, :].astype(jnp.float32)
    rb, s, h = s_raw.shape
    tf = lax.broadcast_in_dim(t, (rb, s, h), (0, 1))
    tok = wtok_ref[0, :] + tf * (wtok_ref[1, :] - wtok_ref[0, :])
    x = s_raw.astype(jnp.float32) + posb_ref[...][None] + tok
    mu = jnp.mean(x, axis=-1, keepdims=True)
    var = jnp.mean(jnp.square(x - mu), axis=-1, keepdims=True)
    y = (x - mu) * lax.rsqrt(var + LN_EPS)
    return y * g_ref[0, :] + b_ref[0, :]


def _post_ln_body(s_ref, t_ref, wtok_ref, posb_ref, g_ref, b_ref, o_ref):
    o_ref[...] = _ln_math(s_ref[...], t_ref, wtok_ref, posb_ref, g_ref, b_ref)


def _post_ln_alias_body(acc_ref, s_ref, t_ref, wtok_ref, posb_ref,
                        g_ref, b_ref, o_ref):
    del acc_ref
    o_ref[...] = _ln_math(s_ref[...], t_ref, wtok_ref, posb_ref, g_ref, b_ref)


def kernel(input_ids, token_type_ids, position_ids, zixing_ids,
           W_word, W_pos, W_tok, W_glyph, W_gfc, b_gfc,
           W_map, b_map, ln_g, ln_b):
    B, S = input_ids.shape
    V, H = W_word.shape
    GV, _ = W_glyph.shape
    N = B * S
    A = W_map[:H]
    G2 = W_map[H:]

    WBLK = 2000
    W_word2 = pl.pallas_call(
        _word_transform_body,
        grid=(V // WBLK,),
        in_specs=[
            pl.BlockSpec((WBLK, H), lambda i: (i, 0)),
            pl.BlockSpec((H, H), lambda i: (0, 0)),
        ],
        out_specs=pl.BlockSpec((WBLK, H), lambda i: (i, 0)),
        out_shape=jax.ShapeDtypeStruct((V, H), jnp.float32),
    )(W_word, A)

    pos3 = position_ids.reshape(1, 1, S).astype(jnp.int32)
    W_glyph2, posb = pl.pallas_call(
        _glyph_transform_body,
        out_shape=(jax.ShapeDtypeStruct((GV, H), jnp.float32),
                   jax.ShapeDtypeStruct((S, H), jnp.float32)),
    )(W_glyph, W_gfc, G2, b_gfc.reshape(1, H), b_map.reshape(1, H),
      pos3, W_pos)

    info = plsc.get_sparse_core_info()
    NC, NS = info.num_cores, info.num_subcores
    NW = NC * NS
    Np = N // NSPLIT
    per_w = Np // NW
    K = 128
    n_chunks = per_w // K
    mesh = plsc.VectorSubcoreMesh(core_axis_name="c", subcore_axis_name="s")

    def _make_gather_sum(part):
        @functools.partial(
            pl.kernel,
            out_type=jax.ShapeDtypeStruct((Np, H), jnp.float32),
            mesh=mesh,
            scratch_types=[
                pltpu.VMEM((K,), jnp.int32),
                pltpu.VMEM((K,), jnp.int32),
                pltpu.VMEM((K, H), jnp.float32),
                pltpu.SemaphoreType.DMA,
            ],
        )
        def _gather_sum(wt_hbm, gt_hbm, idw_hbm, idg_hbm, out_hbm,
                        idw_v, idg_v, rows_v, sem):
            wid = lax.axis_index("s") * NC + lax.axis_index("c")

            @pl.loop(0, n_chunks)
            def _chunk(i):
                base = (wid * n_chunks + i) * K
                gbase = part * Np + base
                pltpu.sync_copy(idw_hbm.at[pl.ds(gbase, K)], idw_v)
                pltpu.sync_copy(idg_hbm.at[pl.ds(gbase, K)], idg_v)
                pltpu.async_copy(wt_hbm.at[idw_v], rows_v, sem).wait()
                pltpu.async_copy(gt_hbm.at[idg_v], rows_v, sem,
                                 add=True).wait()
                pltpu.sync_copy(rows_v, out_hbm.at[pl.ds(base, K)])

        return _gather_sum

    idw = input_ids.reshape(N)
    idg = zixing_ids.reshape(N)
    sums = [_make_gather_sum(p)(W_word2, W_glyph2, idw, idg)
            for p in range(NSPLIT)]

    RB = 16
    Bp = B // NSPLIT
    type3 = token_type_ids.reshape(B, 1, S).astype(jnp.int32)
    common_specs = [
        pl.BlockSpec((RB, S, H), lambda i: (i, 0, 0)),
        pl.BlockSpec((RB, 1, S), lambda i: (i, 0, 0)),
        pl.BlockSpec((2, H), lambda i: (0, 0)),
        pl.BlockSpec((S, H), lambda i: (0, 0)),
        pl.BlockSpec((1, H), lambda i: (0, 0)),
        pl.BlockSpec((1, H), lambda i: (0, 0)),
    ]
    out = None
    for p in range(NSPLIT):
        args = (sums[p].reshape(Bp, S, H),
                lax.slice_in_dim(type3, p * Bp, (p + 1) * Bp),
                W_tok, posb, ln_g.reshape(1, H), ln_b.reshape(1, H))
        blk0 = p * (Bp // RB)
        out_spec = pl.BlockSpec(
            (RB, S, H), functools.partial(lambda b0, i: (i + b0, 0, 0), blk0))
        if p == 0:
            out = pl.pallas_call(
                _post_ln_body,
                grid=(Bp // RB,),
                in_specs=common_specs,
                out_specs=out_spec,
                out_shape=jax.ShapeDtypeStruct((B, S, H), jnp.float32),
            )(*args)
        else:
            out = pl.pallas_call(
                _post_ln_alias_body,
                grid=(Bp // RB,),
                in_specs=[pl.BlockSpec(memory_space=pltpu.MemorySpace.HBM)]
                + common_specs,
                out_specs=out_spec,
                out_shape=jax.ShapeDtypeStruct((B, S, H), jnp.float32),
                input_output_aliases={0: 0},
            )(out, *args)

    return out


# speedup vs baseline: 8.6251x; 1.2975x over previous
import functools

import jax
import jax.numpy as jnp
from jax import lax
from jax.experimental import pallas as pl
from jax.experimental.pallas import tpu as pltpu
from jax.experimental.pallas import tpu_sc as plsc

LN_EPS = 1e-12
NSPLIT = 4


def _word_transform_body(w_ref, a_ref, o_ref):
    o_ref[...] = jnp.dot(w_ref[...], a_ref[...],
                         preferred_element_type=jnp.float32)


def _glyph_transform_body(wg_ref, wgfc_ref, g2_ref, bg_ref, bm_ref,
                          pos_ref, wpos_ref, og_ref, oposb_ref):
    c = jnp.dot(wgfc_ref[...], g2_ref[...], preferred_element_type=jnp.float32)
    og_ref[...] = jnp.dot(wg_ref[...], c, preferred_element_type=jnp.float32)
    bias = jnp.dot(bg_ref[...], g2_ref[...],
                   preferred_element_type=jnp.float32) + bm_ref[...]
    pos = pos_ref[0, 0, :]
    max_pos = wpos_ref.shape[0]
    oh = (pos[:, None] ==
          lax.broadcasted_iota(jnp.int32, (pos.shape[0], max_pos), 1)
          ).astype(jnp.float32)
    oposb_ref[...] = jnp.dot(oh, wpos_ref[...],
                             preferred_element_type=jnp.float32) + bias


def _ln_math(s_raw, t_ref, wtok_ref, posb_ref, g_ref, b_ref):
    t = t_ref[:, 0, :].astype(jnp.float32)
    rb, s, h = s_raw.shape
    tf = lax.broadcast_in_dim(t, (rb, s, h), (0, 1))
    tok = wtok_ref[0, :] + tf * (wtok_ref[1, :] - wtok_ref[0, :])
    x = s_raw.astype(jnp.float32) + posb_ref[...][None] + tok
    mu = jnp.mean(x, axis=-1, keepdims=True)
    var = jnp.mean(jnp.square(x - mu), axis=-1, keepdims=True)
    y = (x - mu) * lax.rsqrt(var + LN_EPS)
    return y * g_ref[0, :] + b_ref[0, :]


def _post_ln_body(s_ref, t_ref, wtok_ref, posb_ref, g_ref, b_ref, o_ref):
    o_ref[...] = _ln_math(s_ref[...], t_ref, wtok_ref, posb_ref, g_ref, b_ref)


def _post_ln_alias_body(acc_ref, s_ref, t_ref, wtok_ref, posb_ref,
                        g_ref, b_ref, o_ref):
    del acc_ref
    o_ref[...] = _ln_math(s_ref[...], t_ref, wtok_ref, posb_ref, g_ref, b_ref)


def kernel(input_ids, token_type_ids, position_ids, zixing_ids,
           W_word, W_pos, W_tok, W_glyph, W_gfc, b_gfc,
           W_map, b_map, ln_g, ln_b):
    B, S = input_ids.shape
    V, H = W_word.shape
    GV, _ = W_glyph.shape
    N = B * S
    A = W_map[:H]
    G2 = W_map[H:]

    WBLK = 4000
    W_word2 = pl.pallas_call(
        _word_transform_body,
        grid=(V // WBLK,),
        in_specs=[
            pl.BlockSpec((WBLK, H), lambda i: (i, 0)),
            pl.BlockSpec((H, H), lambda i: (0, 0)),
        ],
        out_specs=pl.BlockSpec((WBLK, H), lambda i: (i, 0)),
        out_shape=jax.ShapeDtypeStruct((V, H), jnp.float32),
    )(W_word, A)

    pos3 = position_ids.reshape(1, 1, S).astype(jnp.int32)
    W_glyph2, posb = pl.pallas_call(
        _glyph_transform_body,
        out_shape=(jax.ShapeDtypeStruct((GV, H), jnp.float32),
                   jax.ShapeDtypeStruct((S, H), jnp.float32)),
    )(W_glyph, W_gfc, G2, b_gfc.reshape(1, H), b_map.reshape(1, H),
      pos3, W_pos)

    info = plsc.get_sparse_core_info()
    NC, NS = info.num_cores, info.num_subcores
    NW = NC * NS
    Np = N // NSPLIT
    per_w = Np // NW
    K = 128
    n_full = per_w // K
    tail = per_w - n_full * K
    assert n_full % 2 == 0 and tail % 8 == 0
    n_pairs = n_full // 2
    mesh = plsc.VectorSubcoreMesh(core_axis_name="c", subcore_axis_name="s")

    def _make_gather_sum(part):
        @functools.partial(
            pl.kernel,
            out_type=jax.ShapeDtypeStruct((Np, H), jnp.float32),
            mesh=mesh,
            scratch_types=[
                pltpu.VMEM((K,), jnp.int32), pltpu.VMEM((K,), jnp.int32),
                pltpu.VMEM((K,), jnp.int32), pltpu.VMEM((K,), jnp.int32),
                pltpu.VMEM((K, H), jnp.float32),
                pltpu.VMEM((K, H), jnp.float32),
                pltpu.SemaphoreType.DMA, pltpu.SemaphoreType.DMA,
                pltpu.SemaphoreType.DMA, pltpu.SemaphoreType.DMA,
                pltpu.SemaphoreType.DMA, pltpu.SemaphoreType.DMA,
            ],
        )
        def _gather_sum(wt_hbm, gt_hbm, idw_hbm, idg_hbm, out_hbm,
                        idw0, idw1, idg0, idg1, rows0, rows1,
                        semw0, semw1, semg0, semg1, sems0, sems1):
            wid = lax.axis_index("s") * NC + lax.axis_index("c")
            wbase = wid * per_w
            idw_v = (idw0, idw1)
            idg_v = (idg0, idg1)
            rows_v = (rows0, rows1)
            sem_w = (semw0, semw1)
            sem_g = (semg0, semg1)
            sem_s = (sems0, sems1)

            @pl.loop(0, n_pairs)
            def _pair(j):
                i0 = 2 * j
                wdesc = [None, None]
                gdesc = [None, None]
                for b in (0, 1):
                    base = wbase + (i0 + b) * K
                    gbase = part * Np + base
                    pltpu.sync_copy(idw_hbm.at[pl.ds(gbase, K)], idw_v[b])
                    pltpu.sync_copy(idg_hbm.at[pl.ds(gbase, K)], idg_v[b])

                    @pl.when(j > 0)
                    def _drain(b=b, base=base):
                        pltpu.make_async_copy(
                            rows_v[b], out_hbm.at[pl.ds(base, K)],
                            sem_s[b]).wait()

                    wdesc[b] = pltpu.async_copy(
                        wt_hbm.at[idw_v[b]], rows_v[b], sem_w[b])
                for b in (0, 1):
                    wdesc[b].wait()
                    gdesc[b] = pltpu.async_copy(
                        gt_hbm.at[idg_v[b]], rows_v[b], sem_g[b], add=True)
                for b in (0, 1):
                    gdesc[b].wait()
                    base = wbase + (i0 + b) * K
                    pltpu.async_copy(rows_v[b], out_hbm.at[pl.ds(base, K)],
                                     sem_s[b])

            for b in (0, 1):
                base = wbase + (n_full - 2 + b) * K
                pltpu.make_async_copy(rows_v[b], out_hbm.at[pl.ds(base, K)],
                                      sem_s[b]).wait()

            if tail:
                base = wbase + n_full * K
                gbase = part * Np + base
                pltpu.sync_copy(idw_hbm.at[pl.ds(gbase, tail)],
                                idw0.at[pl.ds(0, tail)])
                pltpu.sync_copy(idg_hbm.at[pl.ds(gbase, tail)],
                                idg0.at[pl.ds(0, tail)])
                pltpu.async_copy(wt_hbm.at[idw0.at[pl.ds(0, tail)]],
                                 rows0.at[pl.ds(0, tail)], semw0).wait()
                pltpu.async_copy(gt_hbm.at[idg0.at[pl.ds(0, tail)]],
                                 rows0.at[pl.ds(0, tail)], semg0,
                                 add=True).wait()
                pltpu.sync_copy(rows0.at[pl.ds(0, tail)],
                                out_hbm.at[pl.ds(base, tail)])

        return _gather_sum

    idw = input_ids.reshape(N)
    idg = zixing_ids.reshape(N)
    sums = [_make_gather_sum(p)(W_word2, W_glyph2, idw, idg)
            for p in range(NSPLIT)]

    RB = 16
    Bp = B // NSPLIT
    type3 = token_type_ids.reshape(B, 1, S).astype(jnp.int32)
    common_specs = [
        pl.BlockSpec((RB, S, H), lambda i: (i, 0, 0)),
        pl.BlockSpec((RB, 1, S), lambda i: (i, 0, 0)),
        pl.BlockSpec((2, H), lambda i: (0, 0)),
        pl.BlockSpec((S, H), lambda i: (0, 0)),
        pl.BlockSpec((1, H), lambda i: (0, 0)),
        pl.BlockSpec((1, H), lambda i: (0, 0)),
    ]
    out = None
    for p in range(NSPLIT):
        args = (sums[p].reshape(Bp, S, H),
                lax.slice_in_dim(type3, p * Bp, (p + 1) * Bp),
                W_tok, posb, ln_g.reshape(1, H), ln_b.reshape(1, H))
        blk0 = p * (Bp // RB)
        out_spec = pl.BlockSpec(
            (RB, S, H), functools.partial(lambda b0, i: (i + b0, 0, 0), blk0))
        if p == 0:
            out = pl.pallas_call(
                _post_ln_body,
                grid=(Bp // RB,),
                in_specs=common_specs,
                out_specs=out_spec,
                out_shape=jax.ShapeDtypeStruct((B, S, H), jnp.float32),
            )(*args)
        else:
            out = pl.pallas_call(
                _post_ln_alias_body,
                grid=(Bp // RB,),
                in_specs=[pl.BlockSpec(memory_space=pltpu.MemorySpace.HBM)]
                + common_specs,
                out_specs=out_spec,
                out_shape=jax.ShapeDtypeStruct((B, S, H), jnp.float32),
                input_output_aliases={0: 0},
            )(out, *args)

    return out
